# Initial kernel scaffold; baseline (speedup 1.0000x reference)
import jax
import jax.numpy as jnp
from jax.experimental import pallas as pl


def kernel(x, edge_index, edge_weight, W, b):
    raise NotImplementedError("write your pallas kernel here")


# speedup vs baseline: 9.7343x; 9.7343x over previous
import functools

import jax
import jax.numpy as jnp
from jax import lax
from jax.experimental import pallas as pl
from jax.experimental.pallas import tpu as pltpu
from jax.experimental.pallas import tpu_sc as plsc

N = 10000
E = 320000
D = 128

NC = 2
NS = 16
NW = NC * NS
EPW = E // NW
CH = 80
CHUNKS = EPW // CH
WB = 80
NZCH = N // WB
LANES = 16
VPR = D // LANES


def _linear_body(x_ref, w_ref, b_ref, o_ref):
    o_ref[...] = lax.dot_general(
        x_ref[...], w_ref[...],
        dimension_numbers=(((1,), (1,)), ((), ())),
        preferred_element_type=jnp.float32,
    ) + b_ref[...]


def _linear(x, W, b):
    blk = 1000
    return pl.pallas_call(
        _linear_body,
        grid=(N // blk,),
        in_specs=[
            pl.BlockSpec((blk, D), lambda i: (i, 0)),
            pl.BlockSpec((D, D), lambda i: (0, 0)),
            pl.BlockSpec((1, D), lambda i: (0, 0)),
        ],
        out_specs=pl.BlockSpec((blk, D), lambda i: (i, 0)),
        out_shape=jax.ShapeDtypeStruct((N, D), jnp.float32),
    )(x, W, b.reshape(1, D))


def _sum2_body(p_ref, o_ref):
    o_ref[...] = p_ref[0] + p_ref[1]


def _sum2(partials):
    blk = 1000
    return pl.pallas_call(
        _sum2_body,
        grid=(N // blk,),
        in_specs=[pl.BlockSpec((2, blk, D), lambda i: (0, i, 0))],
        out_specs=pl.BlockSpec((blk, D), lambda i: (i, 0)),
        out_shape=jax.ShapeDtypeStruct((N, D), jnp.float32),
    )(partials)


def _sc_body(h_hbm, packed_hbm, w_hbm, out_hbm,
             packed_v, rows0, rows1, col_b0, col_b1, row_b, w_b0, w_b1,
             acc_shared, sem0, sem1, wsem, ldsem):
    c = lax.axis_index("c")
    s = lax.axis_index("s")
    wid = s * NC + c

    pltpu.async_copy(packed_hbm.at[wid], packed_v, ldsem)

    def zero_buf(i):
        zero = jnp.zeros((LANES,), jnp.float32)
        for k in range(VPR):
            rows0[i, pl.ds(k * LANES, LANES)] = zero
    pl.loop(0, WB)(zero_buf)
    for t in range(pl.cdiv(NZCH, NS)):
        ci = s + NS * t

        @pl.when(ci < NZCH)
        def _():
            pltpu.sync_copy(rows0, acc_shared.at[pl.ds(ci * WB, WB)])

    pltpu.make_async_copy(packed_hbm.at[wid], packed_v, ldsem).wait()

    plsc.subcore_barrier()

    rbufs = (rows0, rows1)
    rsems = (sem0, sem1)
    cbufs = (col_b0, col_b1)
    wbufs = (w_b0, w_b1)

    def unpack_cols(j, dst):
        def body(g):
            p = packed_v[j, pl.ds(g * LANES, LANES)]
            dst[pl.ds(g * LANES, LANES)] = p & jnp.int32(0xFFFF)
        pl.loop(0, CH // LANES)(body)

    def unpack_rows(j, dst):
        def body(g):
            p = packed_v[j, pl.ds(g * LANES, LANES)]
            dst[pl.ds(g * LANES, LANES)] = lax.shift_right_logical(
                p, jnp.int32(16))
        pl.loop(0, CH // LANES)(body)

    def scale_rows(buf, wb):
        def body(g):
            e0 = g * LANES
            wvec = wb[pl.ds(e0, LANES)]
            for l in range(LANES):
                we = wvec[l]
                for k in range(VPR):
                    sl = pl.ds(k * LANES, LANES)
                    buf[e0 + l, sl] = buf[e0 + l, sl] * we
        pl.loop(0, CH // LANES)(body)

    unpack_cols(0, col_b0)
    pltpu.async_copy(h_hbm.at[col_b0], rows0, sem0)
    pltpu.async_copy(w_hbm.at[wid].at[0], w_b0, wsem)

    def step(j):
        for b in range(2):
            jj = j + b

            @pl.when(jj < CHUNKS)
            def _():
                nb = 1 - b

                @pl.when(jj + 1 < CHUNKS)
                def _():
                    unpack_cols(jj + 1, cbufs[nb])
                    pltpu.async_copy(h_hbm.at[cbufs[nb]], rbufs[nb], rsems[nb])
                    pltpu.async_copy(w_hbm.at[wid].at[jj + 1], wbufs[nb], wsem)

                pltpu.make_async_copy(
                    h_hbm.at[cbufs[b]], rbufs[b], rsems[b]).wait()
                pltpu.make_async_copy(
                    w_hbm.at[wid].at[jj], wbufs[b], wsem).wait()
                scale_rows(rbufs[b], wbufs[b])
                unpack_rows(jj, row_b)
                pltpu.sync_copy(rbufs[b], acc_shared.at[row_b], add=True)

    pl.loop(0, CHUNKS + 1, step=2)(step)

    plsc.subcore_barrier()

    for t in range(pl.cdiv(NZCH, NS)):
        ci = s + NS * t

        @pl.when(ci < NZCH)
        def _():
            pltpu.sync_copy(acc_shared.at[pl.ds(ci * WB, WB)],
                            out_hbm.at[c].at[pl.ds(ci * WB, WB)])


def _sc_scatter(h, row, col, w):
    mesh = plsc.VectorSubcoreMesh(core_axis_name="c", subcore_axis_name="s")
    packed = ((row << 16) | col).reshape(NW, CHUNKS, CH)
    w_r = w.reshape(NW, CHUNKS, CH)
    kern = functools.partial(
        pl.kernel,
        out_type=jax.ShapeDtypeStruct((NC, N, D), jnp.float32),
        mesh=mesh,
        scratch_types=[
            pltpu.VMEM((CHUNKS, CH), jnp.int32),
            pltpu.VMEM((CH, D), jnp.float32),
            pltpu.VMEM((CH, D), jnp.float32),
            pltpu.VMEM((CH,), jnp.int32),
            pltpu.VMEM((CH,), jnp.int32),
            pltpu.VMEM((CH,), jnp.int32),
            pltpu.VMEM((CH,), jnp.float32),
            pltpu.VMEM((CH,), jnp.float32),
            pltpu.VMEM_SHARED((N, D), jnp.float32),
            pltpu.SemaphoreType.DMA,
            pltpu.SemaphoreType.DMA,
            pltpu.SemaphoreType.DMA,
            pltpu.SemaphoreType.DMA,
        ],
    )(_sc_body)
    return kern(h, packed, w_r)


@jax.jit
def kernel(x, edge_index, edge_weight, W, b):
    h = _linear(x, W, b)
    partials = _sc_scatter(h, edge_index[0], edge_index[1], edge_weight)
    return _sum2(partials)


# speedup vs baseline: 10.8160x; 1.1111x over previous
import functools

import jax
import jax.numpy as jnp
from jax import lax
from jax.experimental import pallas as pl
from jax.experimental.pallas import tpu as pltpu
from jax.experimental.pallas import tpu_sc as plsc

N = 10000
E = 320000
D = 128

NC = 2
NS = 16
NW = NC * NS
EPW = E // NW
CH = 80
CHUNKS = EPW // CH
WB = 80
NZCH = N // WB
LANES = 16
VPR = D // LANES


def _linear_body(x_ref, w_ref, b_ref, o_ref):
    o_ref[...] = lax.dot_general(
        x_ref[...], w_ref[...],
        dimension_numbers=(((1,), (1,)), ((), ())),
        preferred_element_type=jnp.float32,
    ) + b_ref[...]


def _linear(x, W, b):
    blk = 1000
    return pl.pallas_call(
        _linear_body,
        grid=(N // blk,),
        in_specs=[
            pl.BlockSpec((blk, D), lambda i: (i, 0)),
            pl.BlockSpec((D, D), lambda i: (0, 0)),
            pl.BlockSpec((1, D), lambda i: (0, 0)),
        ],
        out_specs=pl.BlockSpec((blk, D), lambda i: (i, 0)),
        out_shape=jax.ShapeDtypeStruct((N, D), jnp.float32),
    )(x, W, b.reshape(1, D))


_GPERM = []
for _q in range(D // 32):
    for _i in range(LANES):
        _GPERM.append((32 * _q + 2 * _i, 32 * _q + _i))
        _GPERM.append((32 * _q + 2 * _i + 1, 32 * _q + LANES + _i))
_GPERM = tuple(src for _, src in sorted(_GPERM))


def _sum2_body(p_ref, o_ref):
    o_ref[...] = p_ref[0] + p_ref[1]


def _sum2(partials):
    blk = 1000
    return pl.pallas_call(
        _sum2_body,
        grid=(N // blk,),
        in_specs=[pl.BlockSpec((2, blk, D), lambda i: (0, i, 0))],
        out_specs=pl.BlockSpec((blk, D), lambda i: (i, 0)),
        out_shape=jax.ShapeDtypeStruct((N, D), jnp.float32),
    )(partials)


def _sc_body(h_hbm, packed_hbm, w_hbm, out_hbm,
             packed_v, rows0, rows1, rows2, col_b0, col_b1, col_b2,
             row_b0, row_b1, row_b2, w_b0, w_b1, w_b2, acc_shared,
             sem0, sem1, sem2, scsem0, scsem1, scsem2, wsem, ldsem):
    c = lax.axis_index("c")
    s = lax.axis_index("s")
    wid = s * NC + c

    pltpu.async_copy(packed_hbm.at[wid], packed_v, ldsem)

    def zero_buf(i):
        zero = jnp.zeros((LANES,), jnp.float32)
        for k in range(VPR):
            rows0[i, pl.ds(k * LANES, LANES)] = zero
    pl.loop(0, WB)(zero_buf)
    for t in range(pl.cdiv(NZCH, NS)):
        ci = s + NS * t

        @pl.when(ci < NZCH)
        def _():
            pltpu.sync_copy(rows0, acc_shared.at[pl.ds(ci * WB, WB)])

    pltpu.make_async_copy(packed_hbm.at[wid], packed_v, ldsem).wait()

    plsc.subcore_barrier()

    rbufs = (rows0, rows1, rows2)
    rsems = (sem0, sem1, sem2)
    scsems = (scsem0, scsem1, scsem2)
    cbufs = (col_b0, col_b1, col_b2)
    robufs = (row_b0, row_b1, row_b2)
    wbufs = (w_b0, w_b1, w_b2)

    def unpack_cols(j, dst):
        def body(g):
            p = packed_v[j, pl.ds(g * LANES, LANES)]
            dst[pl.ds(g * LANES, LANES)] = p & jnp.int32(0xFFFF)
        pl.loop(0, CH // LANES)(body)

    def unpack_rows(j, dst):
        def body(g):
            p = packed_v[j, pl.ds(g * LANES, LANES)]
            dst[pl.ds(g * LANES, LANES)] = lax.shift_right_logical(
                p, jnp.int32(16))
        pl.loop(0, CH // LANES)(body)

    def scale_rows(buf, wb):
        def body(g):
            e0 = g * LANES
            wvec = wb[pl.ds(e0, LANES)]
            for l in range(LANES):
                we = wvec[l]
                for k in range(VPR):
                    sl = pl.ds(k * LANES, LANES)
                    buf[e0 + l, sl] = buf[e0 + l, sl] * we
        pl.loop(0, CH // LANES)(body)

    unpack_cols(0, col_b0)
    pltpu.async_copy(h_hbm.at[col_b0], rows0, sem0)
    pltpu.async_copy(w_hbm.at[wid].at[0], w_b0, wsem)

    def step(j):
        for b in range(3):
            jj = j + b

            @pl.when(jj < CHUNKS)
            def _():
                nb = (b + 1) % 3

                @pl.when(jj + 1 < CHUNKS)
                def _():
                    @pl.when(jj >= 2)
                    def _():
                        pltpu.make_async_copy(
                            rbufs[nb], acc_shared.at[robufs[nb]],
                            scsems[nb]).wait()
                    unpack_cols(jj + 1, cbufs[nb])
                    pltpu.async_copy(h_hbm.at[cbufs[nb]], rbufs[nb], rsems[nb])
                    pltpu.async_copy(w_hbm.at[wid].at[jj + 1], wbufs[nb], wsem)

                pltpu.make_async_copy(
                    h_hbm.at[cbufs[b]], rbufs[b], rsems[b]).wait()
                pltpu.make_async_copy(
                    w_hbm.at[wid].at[jj], wbufs[b], wsem).wait()
                scale_rows(rbufs[b], wbufs[b])
                unpack_rows(jj, robufs[b])
                pltpu.async_copy(
                    rbufs[b], acc_shared.at[robufs[b]], scsems[b], add=True)

    pl.loop(0, CHUNKS, step=3)(step)

    for b in range(3):
        pltpu.make_async_copy(
            rbufs[b], acc_shared.at[robufs[b]], scsems[b]).wait()

    plsc.subcore_barrier()

    for t in range(pl.cdiv(NZCH, NS)):
        ci = s + NS * t

        @pl.when(ci < NZCH)
        def _():
            pltpu.sync_copy(acc_shared.at[pl.ds(ci * WB, WB)],
                            out_hbm.at[c].at[pl.ds(ci * WB, WB)])


def _sc_scatter(h, row, col, w):
    mesh = plsc.VectorSubcoreMesh(core_axis_name="c", subcore_axis_name="s")
    packed = ((row << 16) | col).reshape(NW, CHUNKS, CH)
    w_r = w.reshape(NW, CHUNKS, CH)
    kern = functools.partial(
        pl.kernel,
        out_type=jax.ShapeDtypeStruct((NC, N, D), jnp.float32),
        mesh=mesh,
        scratch_types=[
            pltpu.VMEM((CHUNKS, CH), jnp.int32),
            pltpu.VMEM((CH, D), jnp.float32),
            pltpu.VMEM((CH, D), jnp.float32),
            pltpu.VMEM((CH, D), jnp.float32),
            pltpu.VMEM((CH,), jnp.int32),
            pltpu.VMEM((CH,), jnp.int32),
            pltpu.VMEM((CH,), jnp.int32),
            pltpu.VMEM((CH,), jnp.int32),
            pltpu.VMEM((CH,), jnp.int32),
            pltpu.VMEM((CH,), jnp.int32),
            pltpu.VMEM((CH,), jnp.float32),
            pltpu.VMEM((CH,), jnp.float32),
            pltpu.VMEM((CH,), jnp.float32),
            pltpu.VMEM_SHARED((N, D), jnp.float32),
            pltpu.SemaphoreType.DMA,
            pltpu.SemaphoreType.DMA,
            pltpu.SemaphoreType.DMA,
            pltpu.SemaphoreType.DMA,
            pltpu.SemaphoreType.DMA,
            pltpu.SemaphoreType.DMA,
            pltpu.SemaphoreType.DMA,
            pltpu.SemaphoreType.DMA,
        ],
    )(_sc_body)
    return kern(h, packed, w_r)


@jax.jit
def kernel(x, edge_index, edge_weight, W, b):
    h = _linear(x, W, b)
    partials = _sc_scatter(h, edge_index[0], edge_index[1], edge_weight)
    return _sum2(partials)
